# baseline (speedup 1.0000x reference)
import functools

import jax
import jax.numpy as jnp
from jax.experimental import pallas as pl
from jax.experimental.pallas import tpu as pltpu
from jax.experimental.pallas import tpu_sc as plsc

H = 16
EPS = 1e-6
NORM_MIN = 1e-12
TOP_K = 2
E = 8
RB = 128
NB = 40
_PREC = jax.lax.Precision.DEFAULT


def _dot(a, b):
    return jax.lax.dot_general(
        a.astype(jnp.bfloat16), b.astype(jnp.bfloat16),
        (((a.ndim - 1,), (0,)), ((), ())),
        preferred_element_type=jnp.float32, precision=_PREC)


def _dot_exact(a, b):
    return jax.lax.dot_general(
        a, b, (((a.ndim - 1,), (0,)), ((), ())),
        preferred_element_type=jnp.float32,
        precision=jax.lax.Precision.HIGHEST)


def _qkv_kernel(x_ref, w_ref, wq_ref, cos_ref, sin_ref, o_ref, *, dh):
    j = pl.program_id(0)
    x = x_ref[...]
    var = jnp.mean(x * x, axis=-1, keepdims=True)
    normed = x * jax.lax.rsqrt(var + EPS) * w_ref[...]
    y = _dot(normed, wq_ref[0])
    c = cos_ref[...]
    s = sin_ref[...]
    hd = dh // 2
    is_qk = j < 2
    for h in range(H):
        a = y[:, h * dh:h * dh + hd]
        b = y[:, h * dh + hd:(h + 1) * dh]
        ra = jnp.where(is_qk, a * c - b * s, a)
        rb = jnp.where(is_qk, b * c + a * s, b)
        o_ref[0, :, h * dh:h * dh + hd] = ra
        o_ref[0, :, h * dh + hd:(h + 1) * dh] = rb


def _attn_kernel(q_ref, k_ref, v_ref, o_ref, *, qb, dh, s_len):
    i = pl.program_id(1)
    q = q_ref[0]
    k = k_ref[0]
    v = v_ref[0]
    scores = jax.lax.dot_general(
        q.astype(jnp.bfloat16), k.astype(jnp.bfloat16),
        (((1,), (1,)), ((), ())),
        preferred_element_type=jnp.float32, precision=_PREC)
    scores = scores * (1.0 / (dh ** 0.5))
    row = i * qb + jax.lax.broadcasted_iota(jnp.int32, (qb, s_len), 0)
    col = jax.lax.broadcasted_iota(jnp.int32, (qb, s_len), 1)
    scores = jnp.where(col <= row, scores, jnp.float32(-1e9))
    m = jnp.max(scores, axis=-1, keepdims=True)
    p = jnp.exp(scores - m)
    p = p / jnp.sum(p, axis=-1, keepdims=True)
    o_ref[...] = _dot(p, v)


def _post_kernel(attn_ref, resid_ref, w2_ref, wo_ref, gw_ref,
                 h1_ref, x2_ref, logits_ref):
    o = _dot(attn_ref[...], wo_ref[...])
    h1 = resid_ref[...] + o
    h1_ref[...] = h1
    var = jnp.mean(h1 * h1, axis=-1, keepdims=True)
    x2 = h1 * jax.lax.rsqrt(var + EPS) * w2_ref[...]
    x2_ref[...] = x2.astype(jnp.bfloat16)
    logits_ref[...] = _dot(x2, gw_ref[...])


def _router_kernel(logits_ref, bias_ref, pos_ref, w_ref, eob_ref, *, s_len):
    z = logits_ref[...]
    m = jnp.max(z, axis=-1, keepdims=True)
    p = jnp.exp(z - m)
    p = p / jnp.sum(p, axis=-1, keepdims=True)
    corrected = p + bias_ref[...]
    col = jax.lax.broadcasted_iota(jnp.int32, (s_len, E), 1)
    c1 = jnp.max(corrected, axis=-1, keepdims=True)
    i1 = jnp.min(jnp.where(corrected == c1, col, E), axis=-1, keepdims=True)
    masked = jnp.where(col == i1, -jnp.inf, corrected)
    c2 = jnp.max(masked, axis=-1, keepdims=True)
    i2 = jnp.min(jnp.where(masked == c2, col, E), axis=-1, keepdims=True)
    rw1 = jnp.sum(jnp.where(col == i1, p, 0.0), axis=-1, keepdims=True)
    rw2 = jnp.sum(jnp.where(col == i2, p, 0.0), axis=-1, keepdims=True)
    denom = jnp.clip(rw1 + rw2, NORM_MIN, None)
    w_ref[...] = jnp.concatenate([rw1 / denom, rw2 / denom], axis=1)

    RR = 2 * s_len // 128
    CC = 128
    pair_e = jnp.concatenate([i1, i2], axis=1).T.reshape(RR, CC)
    eidx = jax.lax.broadcasted_iota(jnp.int32, (E, RR, CC), 0)
    mask = (pair_e[None] == eidx).astype(jnp.float32)
    cidx = jax.lax.broadcasted_iota(jnp.int32, (CC, CC), 0)
    cidx2 = jax.lax.broadcasted_iota(jnp.int32, (CC, CC), 1)
    ut = (cidx <= cidx2).astype(jnp.float32)
    pref = jax.lax.dot_general(
        mask, ut, (((2,), (0,)), ((), ())),
        preferred_element_type=jnp.float32,
        precision=jax.lax.Precision.HIGHEST)
    rowtot = pref[:, :, CC - 1]
    ridx = jax.lax.broadcasted_iota(jnp.int32, (RR, RR), 0)
    ridx2 = jax.lax.broadcasted_iota(jnp.int32, (RR, RR), 1)
    lt = (ridx < ridx2).astype(jnp.float32)
    rowoff = _dot_exact(rowtot, lt)
    rank = rowoff[:, :, None] + pref - 1.0
    counts = rowoff[:, RR - 1:RR] + rowtot[:, RR - 1:RR]
    nblk = jnp.floor((counts + (RB - 1)) * (1.0 / RB))
    eidx1 = jax.lax.broadcasted_iota(jnp.int32, (E, E), 0)
    eidx2 = jax.lax.broadcasted_iota(jnp.int32, (E, E), 1)
    l8 = (eidx2 < eidx1).astype(jnp.float32)
    blk_off = _dot_exact(l8, nblk)
    seg_start = blk_off * float(RB)
    slot = jnp.sum(mask * (seg_start[:, :, None] + rank), axis=0)
    pos2 = slot.reshape(2, s_len).astype(jnp.int32)
    sub = jax.lax.broadcasted_iota(jnp.int32, (2, s_len, 8), 2)
    pos_ref[...] = (pos2[:, :, None] * 8 + sub).reshape(2, 8 * s_len)
    gidx = jax.lax.broadcasted_iota(jnp.int32, (E, NB), 1)
    eob = jnp.sum((gidx >= blk_off.astype(jnp.int32)).astype(jnp.int32),
                  axis=0, keepdims=True) - 1
    eob_ref[...] = eob


def _sc_scatter(x2v, pos, ppad_sub, dsub):
    n_sub = x2v.shape[0]
    win = 128

    @pl.kernel(
        out_type=jax.ShapeDtypeStruct((ppad_sub, dsub), jnp.int32),
        mesh=plsc.VectorSubcoreMesh(core_axis_name="c", subcore_axis_name="s"))
    def k(x_hbm, i_hbm, o_hbm):
        def body(x_vmem, i_vmem):
            pltpu.sync_copy(x_vmem, o_hbm.at[i_vmem.at[0]])

        pltpu.emit_pipeline(
            body,
            grid=(2, n_sub // win),
            in_specs=[
                pl.BlockSpec((win, dsub), lambda kk, i: (i, 0)),
                pl.BlockSpec((1, win), lambda kk, i: (kk, i)),
            ],
            out_specs=[],
            core_axis_name=("c", "s"),
            dimension_semantics=(pltpu.PARALLEL, pltpu.PARALLEL),
        )(x_hbm, i_hbm)

    return k(x2v, pos)


def _moe_kernel(eob_ref, xs_ref, wg_ref, wu_ref, wd_ref, y_ref):
    x = xs_ref[...]
    g = _dot(x, wg_ref[0])
    u = _dot(x, wu_ref[0])
    hh = (g * jax.lax.logistic(g)) * u
    y_ref[...] = _dot(hh, wd_ref[0])


def _sc_gather(ysv, pos_flat, n_idx, dsub):
    win = 128

    @pl.kernel(
        out_type=jax.ShapeDtypeStruct((n_idx, dsub), jnp.float32),
        mesh=plsc.VectorSubcoreMesh(core_axis_name="c", subcore_axis_name="s"))
    def k(y_hbm, i_hbm, o_hbm):
        def body(i_vmem, o_vmem):
            pltpu.sync_copy(y_hbm.at[i_vmem.at[0]], o_vmem)

        pltpu.emit_pipeline(
            body,
            grid=(n_idx // win,),
            in_specs=[pl.BlockSpec((1, win), lambda i: (0, i))],
            out_specs=[pl.BlockSpec((win, dsub), lambda i: (i, 0))],
            core_axis_name=("c", "s"),
            dimension_semantics=(pltpu.PARALLEL,),
        )(i_hbm, o_hbm)

    return k(ysv, pos_flat)


def _combine_kernel(h1_ref, a_ref, b_ref, w_ref, o_ref):
    w = w_ref[...]
    o_ref[...] = (h1_ref[...] + w[:, 0:1] * a_ref[...]
                  + w[:, 1:2] * b_ref[...])


def kernel(hidden_states, ln1_w, ln2_w, Wq, Wk, Wv, Wo, gate_w, bias, cos, sin,
           Wg, Wu, Wd):
    B, S, D = hidden_states.shape
    dh = D // H
    Dff = Wg.shape[-1]
    PPAD = NB * RB
    xf = hidden_states.reshape(S, D)

    def _deinterleave_cols(W):
        return W.reshape(D, H, dh // 2, 2).transpose(0, 1, 3, 2).reshape(D, D)

    Wqkv = jnp.stack([_deinterleave_cols(Wq), _deinterleave_cols(Wk), Wv])
    Wqkv = Wqkv.astype(jnp.bfloat16)
    Wo_b = Wo.astype(jnp.bfloat16)
    Wg_b = Wg.astype(jnp.bfloat16)
    Wu_b = Wu.astype(jnp.bfloat16)
    Wd_b = Wd.astype(jnp.bfloat16)
    cos_h = cos[:, 0::2]
    sin_h = sin[:, 0::2]
    ln1 = ln1_w.reshape(1, D)
    ln2 = ln2_w.reshape(1, D)
    bias2 = bias.reshape(1, E)

    SB = 256
    n_s = S // SB

    qkv = pl.pallas_call(
        functools.partial(_qkv_kernel, dh=dh),
        grid=(3, n_s),
        in_specs=[
            pl.BlockSpec((SB, D), lambda j, s: (s, 0)),
            pl.BlockSpec((1, D), lambda j, s: (0, 0)),
            pl.BlockSpec((1, D, D), lambda j, s: (j, 0, 0)),
            pl.BlockSpec((SB, dh // 2), lambda j, s: (s, 0)),
            pl.BlockSpec((SB, dh // 2), lambda j, s: (s, 0)),
        ],
        out_specs=pl.BlockSpec((1, SB, D), lambda j, s: (j, s, 0)),
        out_shape=jax.ShapeDtypeStruct((3, S, D), jnp.float32),
    )(xf, ln1, Wqkv, cos_h, sin_h)

    QB = 256
    attn = pl.pallas_call(
        functools.partial(_attn_kernel, qb=QB, dh=dh, s_len=S),
        grid=(H, S // QB),
        in_specs=[
            pl.BlockSpec((1, QB, dh), lambda h, i: (0, i, h)),
            pl.BlockSpec((1, S, dh), lambda h, i: (1, 0, h)),
            pl.BlockSpec((1, S, dh), lambda h, i: (2, 0, h)),
        ],
        out_specs=pl.BlockSpec((QB, dh), lambda h, i: (i, h)),
        out_shape=jax.ShapeDtypeStruct((S, D), jnp.float32),
    )(qkv, qkv, qkv)

    h1, x2, logits = pl.pallas_call(
        _post_kernel,
        grid=(n_s,),
        in_specs=[
            pl.BlockSpec((SB, D), lambda s: (s, 0)),
            pl.BlockSpec((SB, D), lambda s: (s, 0)),
            pl.BlockSpec((1, D), lambda s: (0, 0)),
            pl.BlockSpec((D, D), lambda s: (0, 0)),
            pl.BlockSpec((D, E), lambda s: (0, 0)),
        ],
        out_specs=[
            pl.BlockSpec((SB, D), lambda s: (s, 0)),
            pl.BlockSpec((SB, D), lambda s: (s, 0)),
            pl.BlockSpec((SB, E), lambda s: (s, 0)),
        ],
        out_shape=[
            jax.ShapeDtypeStruct((S, D), jnp.float32),
            jax.ShapeDtypeStruct((S, D), jnp.bfloat16),
            jax.ShapeDtypeStruct((S, E), jnp.float32),
        ],
    )(attn, xf, ln2, Wo_b, gate_w)

    pos, wcol, eob = pl.pallas_call(
        functools.partial(_router_kernel, s_len=S),
        grid=(1,),
        in_specs=[
            pl.BlockSpec((S, E), lambda i: (0, 0)),
            pl.BlockSpec((1, E), lambda i: (0, 0)),
        ],
        out_specs=[
            pl.BlockSpec((2, 8 * S), lambda i: (0, 0)),
            pl.BlockSpec((S, 2), lambda i: (0, 0)),
            pl.BlockSpec((1, NB), lambda i: (0, 0)),
        ],
        out_shape=[
            jax.ShapeDtypeStruct((2, 8 * S), jnp.int32),
            jax.ShapeDtypeStruct((S, 2), jnp.float32),
            jax.ShapeDtypeStruct((1, NB), jnp.int32),
        ],
    )(logits, bias2)
    eob1 = eob.reshape(NB)
    DSUB = D // 8

    x2i = jax.lax.bitcast_convert_type(
        x2.reshape(S, D // 2, 2), jnp.int32).reshape(8 * S, DSUB // 2)
    xsi = _sc_scatter(x2i, pos, 8 * PPAD, DSUB // 2)
    xs = jax.lax.bitcast_convert_type(
        xsi.reshape(PPAD, D // 2), jnp.bfloat16).reshape(PPAD, D)

    ys = pl.pallas_call(
        _moe_kernel,
        grid_spec=pltpu.PrefetchScalarGridSpec(
            num_scalar_prefetch=1,
            grid=(NB,),
            in_specs=[
                pl.BlockSpec((RB, D), lambda g, eob_s: (g, 0)),
                pl.BlockSpec((1, D, Dff), lambda g, eob_s: (eob_s[g], 0, 0)),
                pl.BlockSpec((1, D, Dff), lambda g, eob_s: (eob_s[g], 0, 0)),
                pl.BlockSpec((1, Dff, D), lambda g, eob_s: (eob_s[g], 0, 0)),
            ],
            out_specs=pl.BlockSpec((RB, D), lambda g, eob_s: (g, 0)),
        ),
        out_shape=jax.ShapeDtypeStruct((PPAD, D), jnp.float32),
    )(eob1, xs, Wg_b, Wu_b, Wd_b)

    gathered = _sc_gather(ys.reshape(8 * PPAD, DSUB), pos.reshape(1, 16 * S),
                          16 * S, DSUB)
    gathered = gathered.reshape(2 * S, D)

    out = pl.pallas_call(
        _combine_kernel,
        grid=(n_s,),
        in_specs=[
            pl.BlockSpec((SB, D), lambda s: (s, 0)),
            pl.BlockSpec((SB, D), lambda s: (s, 0)),
            pl.BlockSpec((SB, D), lambda s: (s + S // SB, 0)),
            pl.BlockSpec((SB, 2), lambda s: (s, 0)),
        ],
        out_specs=pl.BlockSpec((SB, D), lambda s: (s, 0)),
        out_shape=jax.ShapeDtypeStruct((S, D), jnp.float32),
    )(h1, gathered, gathered, wcol)

    return out.reshape(B, S, D)


# speedup vs baseline: 1.4922x; 1.4922x over previous
import functools

import jax
import jax.numpy as jnp
from jax.experimental import pallas as pl
from jax.experimental.pallas import tpu as pltpu
from jax.experimental.pallas import tpu_sc as plsc

H = 16
EPS = 1e-6
NORM_MIN = 1e-12
TOP_K = 2
E = 8
RB = 128
NB = 40
_PREC = jax.lax.Precision.DEFAULT


def _dot(a, b):
    return jax.lax.dot_general(
        a.astype(jnp.bfloat16), b.astype(jnp.bfloat16),
        (((a.ndim - 1,), (0,)), ((), ())),
        preferred_element_type=jnp.float32, precision=_PREC)


def _dot_exact(a, b):
    return jax.lax.dot_general(
        a, b, (((a.ndim - 1,), (0,)), ((), ())),
        preferred_element_type=jnp.float32,
        precision=jax.lax.Precision.HIGHEST)


def _qkv_kernel(x_ref, w_ref, wq_ref, cos_ref, sin_ref, o_ref, *, dh):
    j = pl.program_id(0)
    x = x_ref[...]
    var = jnp.mean(x * x, axis=-1, keepdims=True)
    normed = x * jax.lax.rsqrt(var + EPS) * w_ref[...]
    y = _dot(normed, wq_ref[0])
    c = cos_ref[...]
    s = sin_ref[...]
    hd = dh // 2
    is_qk = j < 2
    for h in range(H):
        a = y[:, h * dh:h * dh + hd]
        b = y[:, h * dh + hd:(h + 1) * dh]
        ra = jnp.where(is_qk, a * c - b * s, a)
        rb = jnp.where(is_qk, b * c + a * s, b)
        o_ref[0, :, h * dh:h * dh + hd] = ra
        o_ref[0, :, h * dh + hd:(h + 1) * dh] = rb


def _attn_kernel(q_ref, k_ref, v_ref, o_ref, *, qb, dh, s_len):
    i = pl.program_id(1)
    q = q_ref[0]
    k = k_ref[0]
    v = v_ref[0]
    scores = jax.lax.dot_general(
        q.astype(jnp.bfloat16), k.astype(jnp.bfloat16),
        (((1,), (1,)), ((), ())),
        preferred_element_type=jnp.float32, precision=_PREC)
    scores = scores * (1.0 / (dh ** 0.5))
    row = i * qb + jax.lax.broadcasted_iota(jnp.int32, (qb, s_len), 0)
    col = jax.lax.broadcasted_iota(jnp.int32, (qb, s_len), 1)
    scores = jnp.where(col <= row, scores, jnp.float32(-1e9))
    m = jnp.max(scores, axis=-1, keepdims=True)
    p = jnp.exp(scores - m)
    p = p / jnp.sum(p, axis=-1, keepdims=True)
    o_ref[...] = _dot(p, v)


def _post_kernel(attn_ref, resid_ref, w2_ref, wo_ref, gw_ref,
                 h1_ref, x2_ref, logits_ref):
    o = _dot(attn_ref[...], wo_ref[...])
    h1 = resid_ref[...] + o
    h1_ref[...] = h1
    var = jnp.mean(h1 * h1, axis=-1, keepdims=True)
    x2 = h1 * jax.lax.rsqrt(var + EPS) * w2_ref[...]
    x2_ref[...] = x2
    logits_ref[...] = _dot(x2, gw_ref[...])


def _router_kernel(logits_ref, bias_ref, pos_ref, w_ref, eob_ref, *, s_len):
    z = logits_ref[...]
    m = jnp.max(z, axis=-1, keepdims=True)
    p = jnp.exp(z - m)
    p = p / jnp.sum(p, axis=-1, keepdims=True)
    corrected = p + bias_ref[...]
    col = jax.lax.broadcasted_iota(jnp.int32, (s_len, E), 1)
    c1 = jnp.max(corrected, axis=-1, keepdims=True)
    i1 = jnp.min(jnp.where(corrected == c1, col, E), axis=-1, keepdims=True)
    masked = jnp.where(col == i1, -jnp.inf, corrected)
    c2 = jnp.max(masked, axis=-1, keepdims=True)
    i2 = jnp.min(jnp.where(masked == c2, col, E), axis=-1, keepdims=True)
    rw1 = jnp.sum(jnp.where(col == i1, p, 0.0), axis=-1, keepdims=True)
    rw2 = jnp.sum(jnp.where(col == i2, p, 0.0), axis=-1, keepdims=True)
    denom = jnp.clip(rw1 + rw2, NORM_MIN, None)
    w_ref[...] = jnp.concatenate([rw1 / denom, rw2 / denom], axis=1)

    RR = 2 * s_len // 128
    CC = 128
    pair_e = jnp.concatenate([i1, i2], axis=1).T.reshape(RR, CC)
    eidx = jax.lax.broadcasted_iota(jnp.int32, (E, RR, CC), 0)
    mask = (pair_e[None] == eidx).astype(jnp.float32)
    cidx = jax.lax.broadcasted_iota(jnp.int32, (CC, CC), 0)
    cidx2 = jax.lax.broadcasted_iota(jnp.int32, (CC, CC), 1)
    ut = (cidx <= cidx2).astype(jnp.float32)
    pref = jax.lax.dot_general(
        mask, ut, (((2,), (0,)), ((), ())),
        preferred_element_type=jnp.float32,
        precision=jax.lax.Precision.HIGHEST)
    rowtot = pref[:, :, CC - 1]
    ridx = jax.lax.broadcasted_iota(jnp.int32, (RR, RR), 0)
    ridx2 = jax.lax.broadcasted_iota(jnp.int32, (RR, RR), 1)
    lt = (ridx < ridx2).astype(jnp.float32)
    rowoff = _dot_exact(rowtot, lt)
    rank = rowoff[:, :, None] + pref - 1.0
    counts = rowoff[:, RR - 1:RR] + rowtot[:, RR - 1:RR]
    nblk = jnp.floor((counts + (RB - 1)) * (1.0 / RB))
    eidx1 = jax.lax.broadcasted_iota(jnp.int32, (E, E), 0)
    eidx2 = jax.lax.broadcasted_iota(jnp.int32, (E, E), 1)
    l8 = (eidx2 < eidx1).astype(jnp.float32)
    blk_off = _dot_exact(l8, nblk)
    seg_start = blk_off * float(RB)
    slot = jnp.sum(mask * (seg_start[:, :, None] + rank), axis=0)
    pos_ref[...] = slot.reshape(2, s_len).astype(jnp.int32)
    gidx = jax.lax.broadcasted_iota(jnp.int32, (E, NB), 1)
    eob = jnp.sum((gidx >= blk_off.astype(jnp.int32)).astype(jnp.int32),
                  axis=0, keepdims=True) - 1
    eob_ref[...] = eob


def _sc_scatter(x2, pos, ppad, np_, pw):
    s_len = x2.shape[0]
    win = 128

    @pl.kernel(
        out_type=jax.ShapeDtypeStruct((np_, ppad, pw), jnp.float32),
        mesh=plsc.VectorSubcoreMesh(core_axis_name="c", subcore_axis_name="s"))
    def k(x_hbm, i_hbm, o_hbm):
        def body(x_vmem, i_vmem, *, p):
            pltpu.sync_copy(x_vmem, o_hbm.at[p].at[i_vmem.at[0]])

        for p in range(np_):
            pltpu.emit_pipeline(
                functools.partial(body, p=p),
                grid=(2, s_len // win),
                in_specs=[
                    pl.BlockSpec((win, pw), lambda kk, i, p=p: (i, p)),
                    pl.BlockSpec((1, win), lambda kk, i: (kk, i)),
                ],
                out_specs=[],
                core_axis_name=("c", "s"),
                dimension_semantics=(pltpu.PARALLEL, pltpu.PARALLEL),
            )(x_hbm, i_hbm)

    return k(x2, pos)


def _moe_kernel(eob_ref, *refs, np_, pw):
    xs_refs = refs[:np_]
    wg_ref, wu_ref, wd_ref, y_ref = refs[np_:]
    g = None
    u = None
    for p in range(np_):
        xp = xs_refs[p][0]
        gp = _dot(xp, wg_ref[0, p * pw:(p + 1) * pw, :])
        up = _dot(xp, wu_ref[0, p * pw:(p + 1) * pw, :])
        g = gp if g is None else g + gp
        u = up if u is None else u + up
    hh = (g * jax.lax.logistic(g)) * u
    y = _dot(hh, wd_ref[0])
    for p in range(np_):
        y_ref[p] = y[:, p * pw:(p + 1) * pw]


def _sc_gather(ys3, pos, np_, pw):
    s_len = pos.shape[1]
    win = 128

    @pl.kernel(
        out_type=jax.ShapeDtypeStruct((np_, 2 * s_len, pw), jnp.float32),
        mesh=plsc.VectorSubcoreMesh(core_axis_name="c", subcore_axis_name="s"))
    def k(y_hbm, i_hbm, o_hbm):
        def body(i_vmem, o_vmem, *, p):
            pltpu.sync_copy(y_hbm.at[p].at[i_vmem.at[0]], o_vmem.at[0])

        for p in range(np_):
            pltpu.emit_pipeline(
                functools.partial(body, p=p),
                grid=(2, s_len // win),
                in_specs=[pl.BlockSpec((1, win), lambda kk, i: (kk, i))],
                out_specs=[pl.BlockSpec(
                    (1, win, pw),
                    lambda kk, i, p=p, nw=s_len // win: (p, kk * nw + i, 0))],
                core_axis_name=("c", "s"),
                dimension_semantics=(pltpu.PARALLEL, pltpu.PARALLEL),
            )(i_hbm, o_hbm)

    return k(ys3, pos)


def _combine_kernel(h1_ref, w_ref, *refs, np_, pw):
    a_refs = refs[:np_]
    b_refs = refs[np_:2 * np_]
    o_ref = refs[2 * np_]
    w = w_ref[...]
    w0 = w[:, 0:1]
    w1 = w[:, 1:2]
    for p in range(np_):
        sl = slice(p * pw, (p + 1) * pw)
        o_ref[:, sl] = (h1_ref[:, sl] + w0 * a_refs[p][0] + w1 * b_refs[p][0])


def kernel(hidden_states, ln1_w, ln2_w, Wq, Wk, Wv, Wo, gate_w, bias, cos, sin,
           Wg, Wu, Wd):
    B, S, D = hidden_states.shape
    dh = D // H
    Dff = Wg.shape[-1]
    PPAD = NB * RB
    xf = hidden_states.reshape(S, D)

    def _deinterleave_cols(W):
        return W.reshape(D, H, dh // 2, 2).transpose(0, 1, 3, 2).reshape(D, D)

    Wqkv = jnp.stack([_deinterleave_cols(Wq), _deinterleave_cols(Wk), Wv])
    Wqkv = Wqkv.astype(jnp.bfloat16)
    Wo_b = Wo.astype(jnp.bfloat16)
    Wg_b = Wg.astype(jnp.bfloat16)
    Wu_b = Wu.astype(jnp.bfloat16)
    Wd_b = Wd.astype(jnp.bfloat16)
    cos_h = cos[:, 0::2]
    sin_h = sin[:, 0::2]
    ln1 = ln1_w.reshape(1, D)
    ln2 = ln2_w.reshape(1, D)
    bias2 = bias.reshape(1, E)

    SB = 256
    n_s = S // SB

    qkv = pl.pallas_call(
        functools.partial(_qkv_kernel, dh=dh),
        grid=(3, n_s),
        in_specs=[
            pl.BlockSpec((SB, D), lambda j, s: (s, 0)),
            pl.BlockSpec((1, D), lambda j, s: (0, 0)),
            pl.BlockSpec((1, D, D), lambda j, s: (j, 0, 0)),
            pl.BlockSpec((SB, dh // 2), lambda j, s: (s, 0)),
            pl.BlockSpec((SB, dh // 2), lambda j, s: (s, 0)),
        ],
        out_specs=pl.BlockSpec((1, SB, D), lambda j, s: (j, s, 0)),
        out_shape=jax.ShapeDtypeStruct((3, S, D), jnp.float32),
    )(xf, ln1, Wqkv, cos_h, sin_h)

    QB = 256
    attn = pl.pallas_call(
        functools.partial(_attn_kernel, qb=QB, dh=dh, s_len=S),
        grid=(H, S // QB),
        in_specs=[
            pl.BlockSpec((1, QB, dh), lambda h, i: (0, i, h)),
            pl.BlockSpec((1, S, dh), lambda h, i: (1, 0, h)),
            pl.BlockSpec((1, S, dh), lambda h, i: (2, 0, h)),
        ],
        out_specs=pl.BlockSpec((QB, dh), lambda h, i: (i, h)),
        out_shape=jax.ShapeDtypeStruct((S, D), jnp.float32),
    )(qkv, qkv, qkv)

    h1, x2, logits = pl.pallas_call(
        _post_kernel,
        grid=(n_s,),
        in_specs=[
            pl.BlockSpec((SB, D), lambda s: (s, 0)),
            pl.BlockSpec((SB, D), lambda s: (s, 0)),
            pl.BlockSpec((1, D), lambda s: (0, 0)),
            pl.BlockSpec((D, D), lambda s: (0, 0)),
            pl.BlockSpec((D, E), lambda s: (0, 0)),
        ],
        out_specs=[
            pl.BlockSpec((SB, D), lambda s: (s, 0)),
            pl.BlockSpec((SB, D), lambda s: (s, 0)),
            pl.BlockSpec((SB, E), lambda s: (s, 0)),
        ],
        out_shape=[
            jax.ShapeDtypeStruct((S, D), jnp.float32),
            jax.ShapeDtypeStruct((S, D), jnp.float32),
            jax.ShapeDtypeStruct((S, E), jnp.float32),
        ],
    )(attn, xf, ln2, Wo_b, gate_w)

    pos, wcol, eob = pl.pallas_call(
        functools.partial(_router_kernel, s_len=S),
        grid=(1,),
        in_specs=[
            pl.BlockSpec((S, E), lambda i: (0, 0)),
            pl.BlockSpec((1, E), lambda i: (0, 0)),
        ],
        out_specs=[
            pl.BlockSpec((2, S), lambda i: (0, 0)),
            pl.BlockSpec((S, 2), lambda i: (0, 0)),
            pl.BlockSpec((1, NB), lambda i: (0, 0)),
        ],
        out_shape=[
            jax.ShapeDtypeStruct((2, S), jnp.int32),
            jax.ShapeDtypeStruct((S, 2), jnp.float32),
            jax.ShapeDtypeStruct((1, NB), jnp.int32),
        ],
    )(logits, bias2)
    eob1 = eob.reshape(NB)
    NP = 8
    PW = D // NP

    xs3 = _sc_scatter(x2, pos, PPAD, NP, PW)

    xs_specs = [
        pl.BlockSpec((1, RB, PW), lambda g, eob_s, p=p: (p, g, 0))
        for p in range(NP)
    ]
    ys3 = pl.pallas_call(
        functools.partial(_moe_kernel, np_=NP, pw=PW),
        grid_spec=pltpu.PrefetchScalarGridSpec(
            num_scalar_prefetch=1,
            grid=(NB,),
            in_specs=xs_specs + [
                pl.BlockSpec((1, D, Dff), lambda g, eob_s: (eob_s[g], 0, 0)),
                pl.BlockSpec((1, D, Dff), lambda g, eob_s: (eob_s[g], 0, 0)),
                pl.BlockSpec((1, Dff, D), lambda g, eob_s: (eob_s[g], 0, 0)),
            ],
            out_specs=pl.BlockSpec((NP, RB, PW), lambda g, eob_s: (0, g, 0)),
        ),
        out_shape=jax.ShapeDtypeStruct((NP, PPAD, PW), jnp.float32),
    )(eob1, *([xs3] * NP), Wg_b, Wu_b, Wd_b)

    gathered = _sc_gather(ys3, pos, NP, PW)

    a_specs = [
        pl.BlockSpec((1, SB, PW), lambda s, p=p: (p, s, 0))
        for p in range(NP)
    ]
    b_specs = [
        pl.BlockSpec((1, SB, PW), lambda s, p=p, o=S // SB: (p, s + o, 0))
        for p in range(NP)
    ]
    out = pl.pallas_call(
        functools.partial(_combine_kernel, np_=NP, pw=PW),
        grid=(n_s,),
        in_specs=[
            pl.BlockSpec((SB, D), lambda s: (s, 0)),
            pl.BlockSpec((SB, 2), lambda s: (s, 0)),
        ] + a_specs + b_specs,
        out_specs=pl.BlockSpec((SB, D), lambda s: (s, 0)),
        out_shape=jax.ShapeDtypeStruct((S, D), jnp.float32),
    )(h1, wcol, *([gathered] * NP), *([gathered] * NP))

    return out.reshape(B, S, D)


# speedup vs baseline: 1.9619x; 1.3148x over previous
import functools

import jax
import jax.numpy as jnp
from jax.experimental import pallas as pl
from jax.experimental.pallas import tpu as pltpu
from jax.experimental.pallas import tpu_sc as plsc

H = 16
EPS = 1e-6
NORM_MIN = 1e-12
TOP_K = 2
E = 8
RB = 128
NB = 40
_PREC = jax.lax.Precision.DEFAULT


def _dot(a, b):
    return jax.lax.dot_general(
        a.astype(jnp.bfloat16), b.astype(jnp.bfloat16),
        (((a.ndim - 1,), (0,)), ((), ())),
        preferred_element_type=jnp.float32, precision=_PREC)


def _dot_exact(a, b):
    return jax.lax.dot_general(
        a, b, (((a.ndim - 1,), (0,)), ((), ())),
        preferred_element_type=jnp.float32,
        precision=jax.lax.Precision.HIGHEST)


def _proj_kernel(x_ref, w_ref, wq_ref, cos_ref, sin_ref, o_ref, *, dh, rope):
    x = x_ref[...]
    var = jnp.mean(x * x, axis=-1, keepdims=True)
    normed = x * jax.lax.rsqrt(var + EPS) * w_ref[...]
    y = _dot(normed, wq_ref[...])
    if not rope:
        o_ref[...] = y
        return
    c = cos_ref[...]
    s = sin_ref[...]
    lane = jax.lax.broadcasted_iota(jnp.int32, (y.shape[0], dh), 1)
    even = (lane % 2) == 0
    for h in range(H):
        yh = y[:, h * dh:(h + 1) * dh]
        rot = jnp.where(even, -jnp.roll(yh, -1, axis=1), jnp.roll(yh, 1, axis=1))
        o_ref[:, h * dh:(h + 1) * dh] = yh * c + rot * s


def _attn_kernel(q_ref, k_ref, v_ref, o_ref, *, qb, dh, s_len):
    i = pl.program_id(1)
    q = q_ref[...]
    k = k_ref[...]
    v = v_ref[...]
    scores = jax.lax.dot_general(
        q.astype(jnp.bfloat16), k.astype(jnp.bfloat16),
        (((1,), (1,)), ((), ())),
        preferred_element_type=jnp.float32, precision=_PREC)
    scores = scores * (1.0 / (dh ** 0.5))
    row = i * qb + jax.lax.broadcasted_iota(jnp.int32, (qb, s_len), 0)
    col = jax.lax.broadcasted_iota(jnp.int32, (qb, s_len), 1)
    scores = jnp.where(col <= row, scores, jnp.float32(-1e9))
    m = jnp.max(scores, axis=-1, keepdims=True)
    p = jnp.exp(scores - m)
    p = p / jnp.sum(p, axis=-1, keepdims=True)
    o_ref[...] = _dot(p, v)


def _post_kernel(attn_ref, resid_ref, w2_ref, wo_ref, gw_ref,
                 h1_ref, x2_ref, logits_ref):
    o = _dot(attn_ref[...], wo_ref[...])
    h1 = resid_ref[...] + o
    h1_ref[...] = h1
    var = jnp.mean(h1 * h1, axis=-1, keepdims=True)
    x2 = h1 * jax.lax.rsqrt(var + EPS) * w2_ref[...]
    x2_ref[...] = x2
    logits_ref[...] = _dot(x2, gw_ref[...])


def _router_kernel(logits_ref, bias_ref, pos_ref, w_ref, eob_ref, *, s_len):
    z = logits_ref[...]
    m = jnp.max(z, axis=-1, keepdims=True)
    p = jnp.exp(z - m)
    p = p / jnp.sum(p, axis=-1, keepdims=True)
    corrected = p + bias_ref[...]
    col = jax.lax.broadcasted_iota(jnp.int32, (s_len, E), 1)
    c1 = jnp.max(corrected, axis=-1, keepdims=True)
    i1 = jnp.min(jnp.where(corrected == c1, col, E), axis=-1, keepdims=True)
    masked = jnp.where(col == i1, -jnp.inf, corrected)
    c2 = jnp.max(masked, axis=-1, keepdims=True)
    i2 = jnp.min(jnp.where(masked == c2, col, E), axis=-1, keepdims=True)
    rw1 = jnp.sum(jnp.where(col == i1, p, 0.0), axis=-1, keepdims=True)
    rw2 = jnp.sum(jnp.where(col == i2, p, 0.0), axis=-1, keepdims=True)
    denom = jnp.clip(rw1 + rw2, NORM_MIN, None)
    w_ref[...] = jnp.concatenate([rw1 / denom, rw2 / denom], axis=1)

    RR = 2 * s_len // 128
    CC = 128
    pair_e = jnp.concatenate([i1, i2], axis=1).T.reshape(RR, CC)
    eidx = jax.lax.broadcasted_iota(jnp.int32, (E, RR, CC), 0)
    mask = (pair_e[None] == eidx).astype(jnp.float32)
    cidx = jax.lax.broadcasted_iota(jnp.int32, (CC, CC), 0)
    cidx2 = jax.lax.broadcasted_iota(jnp.int32, (CC, CC), 1)
    ut = (cidx <= cidx2).astype(jnp.float32)
    pref = jax.lax.dot_general(
        mask, ut, (((2,), (0,)), ((), ())),
        preferred_element_type=jnp.float32,
        precision=jax.lax.Precision.HIGHEST)
    rowtot = pref[:, :, CC - 1]
    ridx = jax.lax.broadcasted_iota(jnp.int32, (RR, RR), 0)
    ridx2 = jax.lax.broadcasted_iota(jnp.int32, (RR, RR), 1)
    lt = (ridx < ridx2).astype(jnp.float32)
    rowoff = _dot_exact(rowtot, lt)
    rank = rowoff[:, :, None] + pref - 1.0
    counts = rowoff[:, RR - 1:RR] + rowtot[:, RR - 1:RR]
    nblk = jnp.floor((counts + (RB - 1)) * (1.0 / RB))
    eidx1 = jax.lax.broadcasted_iota(jnp.int32, (E, E), 0)
    eidx2 = jax.lax.broadcasted_iota(jnp.int32, (E, E), 1)
    l8 = (eidx2 < eidx1).astype(jnp.float32)
    blk_off = _dot_exact(l8, nblk)
    seg_start = blk_off * float(RB)
    slot = jnp.sum(mask * (seg_start[:, :, None] + rank), axis=0)
    pos_ref[...] = slot.reshape(2, s_len).astype(jnp.int32)
    gidx = jax.lax.broadcasted_iota(jnp.int32, (E, NB), 1)
    eob = jnp.sum((gidx >= blk_off.astype(jnp.int32)).astype(jnp.int32),
                  axis=0, keepdims=True) - 1
    eob_ref[...] = eob


def _sc_scatter(x2, pos, ppad, np_, pw):
    s_len = x2.shape[0]
    win = 128

    @pl.kernel(
        out_type=jax.ShapeDtypeStruct((np_, ppad, pw), jnp.float32),
        mesh=plsc.VectorSubcoreMesh(core_axis_name="c", subcore_axis_name="s"))
    def k(x_hbm, i_hbm, o_hbm):
        def body(x_vmem, i_vmem, *, p):
            pltpu.sync_copy(x_vmem, o_hbm.at[p].at[i_vmem.at[0]])

        for p in range(np_):
            pltpu.emit_pipeline(
                functools.partial(body, p=p),
                grid=(2, s_len // win),
                in_specs=[
                    pl.BlockSpec((win, pw), lambda kk, i, p=p: (i, p)),
                    pl.BlockSpec((1, win), lambda kk, i: (kk, i)),
                ],
                out_specs=[],
                core_axis_name=("c", "s"),
                dimension_semantics=(pltpu.PARALLEL, pltpu.PARALLEL),
            )(x_hbm, i_hbm)

    return k(x2, pos)


def _moe_kernel(eob_ref, *refs, np_, pw):
    xs_refs = refs[:np_]
    wg_ref, wu_ref, wd_ref, y_ref = refs[np_:]
    g = None
    u = None
    for p in range(np_):
        xp = xs_refs[p][0]
        gp = _dot(xp, wg_ref[0, p * pw:(p + 1) * pw, :])
        up = _dot(xp, wu_ref[0, p * pw:(p + 1) * pw, :])
        g = gp if g is None else g + gp
        u = up if u is None else u + up
    hh = (g * jax.lax.logistic(g)) * u
    y = _dot(hh, wd_ref[0])
    for p in range(np_):
        y_ref[p] = y[:, p * pw:(p + 1) * pw]


def _sc_gather(ys3, pos, np_, pw):
    s_len = pos.shape[1]
    win = 128

    @pl.kernel(
        out_type=jax.ShapeDtypeStruct((np_, 2 * s_len, pw), jnp.float32),
        mesh=plsc.VectorSubcoreMesh(core_axis_name="c", subcore_axis_name="s"))
    def k(y_hbm, i_hbm, o_hbm):
        def body(i_vmem, o_vmem, *, p):
            pltpu.sync_copy(y_hbm.at[p].at[i_vmem.at[0]], o_vmem.at[0])

        for p in range(np_):
            pltpu.emit_pipeline(
                functools.partial(body, p=p),
                grid=(2, s_len // win),
                in_specs=[pl.BlockSpec((1, win), lambda kk, i: (kk, i))],
                out_specs=[pl.BlockSpec(
                    (1, win, pw),
                    lambda kk, i, p=p, nw=s_len // win: (p, kk * nw + i, 0))],
                core_axis_name=("c", "s"),
                dimension_semantics=(pltpu.PARALLEL, pltpu.PARALLEL),
            )(i_hbm, o_hbm)

    return k(ys3, pos)


def _combine_kernel(h1_ref, w_ref, *refs, np_, pw):
    a_refs = refs[:np_]
    b_refs = refs[np_:2 * np_]
    o_ref = refs[2 * np_]
    w = w_ref[...]
    w0 = w[:, 0:1]
    w1 = w[:, 1:2]
    for p in range(np_):
        sl = slice(p * pw, (p + 1) * pw)
        o_ref[:, sl] = (h1_ref[:, sl] + w0 * a_refs[p][0] + w1 * b_refs[p][0])


def kernel(hidden_states, ln1_w, ln2_w, Wq, Wk, Wv, Wo, gate_w, bias, cos, sin,
           Wg, Wu, Wd):
    B, S, D = hidden_states.shape
    dh = D // H
    Dff = Wg.shape[-1]
    PPAD = NB * RB
    xf = hidden_states.reshape(S, D)

    ln1 = ln1_w.reshape(1, D)
    ln2 = ln2_w.reshape(1, D)
    bias2 = bias.reshape(1, E)

    SB = 256
    n_s = S // SB

    def _proj(W, rope):
        return pl.pallas_call(
            functools.partial(_proj_kernel, dh=dh, rope=rope),
            grid=(n_s,),
            in_specs=[
                pl.BlockSpec((SB, D), lambda s: (s, 0)),
                pl.BlockSpec((1, D), lambda s: (0, 0)),
                pl.BlockSpec((D, D), lambda s: (0, 0)),
                pl.BlockSpec((SB, dh), lambda s: (s, 0)),
                pl.BlockSpec((SB, dh), lambda s: (s, 0)),
            ],
            out_specs=pl.BlockSpec((SB, D), lambda s: (s, 0)),
            out_shape=jax.ShapeDtypeStruct((S, D), jnp.float32),
        )(xf, ln1, W, cos, sin)

    q = _proj(Wq, True)
    k = _proj(Wk, True)
    v = _proj(Wv, False)

    QB = 256
    attn = pl.pallas_call(
        functools.partial(_attn_kernel, qb=QB, dh=dh, s_len=S),
        grid=(H, S // QB),
        in_specs=[
            pl.BlockSpec((QB, dh), lambda h, i: (i, h)),
            pl.BlockSpec((S, dh), lambda h, i: (0, h)),
            pl.BlockSpec((S, dh), lambda h, i: (0, h)),
        ],
        out_specs=pl.BlockSpec((QB, dh), lambda h, i: (i, h)),
        out_shape=jax.ShapeDtypeStruct((S, D), jnp.float32),
    )(q, k, v)

    h1, x2, logits = pl.pallas_call(
        _post_kernel,
        grid=(n_s,),
        in_specs=[
            pl.BlockSpec((SB, D), lambda s: (s, 0)),
            pl.BlockSpec((SB, D), lambda s: (s, 0)),
            pl.BlockSpec((1, D), lambda s: (0, 0)),
            pl.BlockSpec((D, D), lambda s: (0, 0)),
            pl.BlockSpec((D, E), lambda s: (0, 0)),
        ],
        out_specs=[
            pl.BlockSpec((SB, D), lambda s: (s, 0)),
            pl.BlockSpec((SB, D), lambda s: (s, 0)),
            pl.BlockSpec((SB, E), lambda s: (s, 0)),
        ],
        out_shape=[
            jax.ShapeDtypeStruct((S, D), jnp.float32),
            jax.ShapeDtypeStruct((S, D), jnp.float32),
            jax.ShapeDtypeStruct((S, E), jnp.float32),
        ],
    )(attn, xf, ln2, Wo, gate_w)

    pos, wcol, eob = pl.pallas_call(
        functools.partial(_router_kernel, s_len=S),
        grid=(1,),
        in_specs=[
            pl.BlockSpec((S, E), lambda i: (0, 0)),
            pl.BlockSpec((1, E), lambda i: (0, 0)),
        ],
        out_specs=[
            pl.BlockSpec((2, S), lambda i: (0, 0)),
            pl.BlockSpec((S, 2), lambda i: (0, 0)),
            pl.BlockSpec((1, NB), lambda i: (0, 0)),
        ],
        out_shape=[
            jax.ShapeDtypeStruct((2, S), jnp.int32),
            jax.ShapeDtypeStruct((S, 2), jnp.float32),
            jax.ShapeDtypeStruct((1, NB), jnp.int32),
        ],
    )(logits, bias2)
    eob1 = eob.reshape(NB)
    NP = 8
    PW = D // NP

    xs3 = _sc_scatter(x2, pos, PPAD, NP, PW)

    xs_specs = [
        pl.BlockSpec((1, RB, PW), lambda g, eob_s, p=p: (p, g, 0))
        for p in range(NP)
    ]
    ys3 = pl.pallas_call(
        functools.partial(_moe_kernel, np_=NP, pw=PW),
        grid_spec=pltpu.PrefetchScalarGridSpec(
            num_scalar_prefetch=1,
            grid=(NB,),
            in_specs=xs_specs + [
                pl.BlockSpec((1, D, Dff), lambda g, eob_s: (eob_s[g], 0, 0)),
                pl.BlockSpec((1, D, Dff), lambda g, eob_s: (eob_s[g], 0, 0)),
                pl.BlockSpec((1, Dff, D), lambda g, eob_s: (eob_s[g], 0, 0)),
            ],
            out_specs=pl.BlockSpec((NP, RB, PW), lambda g, eob_s: (0, g, 0)),
        ),
        out_shape=jax.ShapeDtypeStruct((NP, PPAD, PW), jnp.float32),
    )(eob1, *([xs3] * NP), Wg, Wu, Wd)

    gathered = _sc_gather(ys3, pos, NP, PW)

    a_specs = [
        pl.BlockSpec((1, SB, PW), lambda s, p=p: (p, s, 0))
        for p in range(NP)
    ]
    b_specs = [
        pl.BlockSpec((1, SB, PW), lambda s, p=p, o=S // SB: (p, s + o, 0))
        for p in range(NP)
    ]
    out = pl.pallas_call(
        functools.partial(_combine_kernel, np_=NP, pw=PW),
        grid=(n_s,),
        in_specs=[
            pl.BlockSpec((SB, D), lambda s: (s, 0)),
            pl.BlockSpec((SB, 2), lambda s: (s, 0)),
        ] + a_specs + b_specs,
        out_specs=pl.BlockSpec((SB, D), lambda s: (s, 0)),
        out_shape=jax.ShapeDtypeStruct((S, D), jnp.float32),
    )(h1, wcol, *([gathered] * NP), *([gathered] * NP))

    return out.reshape(B, S, D)
